# Initial kernel scaffold; baseline (speedup 1.0000x reference)
import functools

import jax
import jax.numpy as jnp
from jax import lax
from jax.experimental import pallas as pl
from jax.experimental.pallas import tpu as pltpu
from jax.experimental.pallas import tpu_sc as plsc

_HIDDEN = 128
_NVREG = _HIDDEN // 16
_CHUNK = 128


@functools.partial(jax.jit, static_argnums=(4, 5))
def _embed(idx, tf, vocab_table, type_table, n_rows, n_workers):
    per_w = n_rows // n_workers
    n_chunks = per_w // _CHUNK
    nc = plsc.get_sparse_core_info().num_cores

    def body(idx_hbm, tf_hbm, vt_hbm, tt_hbm, out_hbm,
             idx_v, tf_v, rows_v, tt_v, sem):
        wid = lax.axis_index("s") * nc + lax.axis_index("c")
        base = wid * per_w
        pltpu.sync_copy(tt_hbm, tt_v)
        t0 = [tt_v[0, pl.ds(16 * k, 16)] for k in range(_NVREG)]
        dt = [tt_v[1, pl.ds(16 * k, 16)] - t0[k] for k in range(_NVREG)]

        def chunk_body(c, carry):
            off = base + c * _CHUNK
            pltpu.sync_copy(idx_hbm.at[pl.ds(off, _CHUNK)], idx_v)
            pltpu.sync_copy(tf_hbm.at[pl.ds(off, _CHUNK)], tf_v)
            pltpu.async_copy(vt_hbm.at[idx_v], rows_v, sem).wait()

            def row_body(r, rcarry):
                tsp = plsc.load_gather(tf_v, [jnp.full((16,), r, jnp.int32)])
                for k in range(_NVREG):
                    sl = pl.ds(16 * k, 16)
                    rows_v[r, sl] = rows_v[r, sl] + (t0[k] + tsp * dt[k])
                return rcarry

            lax.fori_loop(0, _CHUNK, row_body, 0)
            pltpu.sync_copy(rows_v, out_hbm.at[pl.ds(off, _CHUNK)])
            return carry

        lax.fori_loop(0, n_chunks, chunk_body, 0)

    return pl.kernel(
        body,
        out_type=jax.ShapeDtypeStruct((n_rows, _HIDDEN), jnp.float32),
        mesh=plsc.VectorSubcoreMesh(core_axis_name="c", subcore_axis_name="s"),
        scratch_types=[
            pltpu.VMEM((_CHUNK,), jnp.int32),
            pltpu.VMEM((_CHUNK,), jnp.float32),
            pltpu.VMEM((_CHUNK, _HIDDEN), jnp.float32),
            pltpu.VMEM((2, _HIDDEN), jnp.float32),
            pltpu.SemaphoreType.DMA,
        ],
    )(idx, tf, vocab_table, type_table)


def kernel(vocab, type, vocab_table, type_table):
    b, l = vocab.shape
    n_rows = b * l
    info = plsc.get_sparse_core_info()
    n_workers = info.num_cores * info.num_subcores
    idx = vocab.reshape(n_rows)
    tf = type.reshape(n_rows).astype(jnp.float32)
    out = _embed(idx, tf, vocab_table, type_table, n_rows, n_workers)
    return out.reshape(b, l, _HIDDEN)


# speedup vs baseline: 4.0129x; 4.0129x over previous
import functools

import jax
import jax.numpy as jnp
from jax import lax
from jax.experimental import pallas as pl
from jax.experimental.pallas import tpu as pltpu
from jax.experimental.pallas import tpu_sc as plsc

_HIDDEN = 128
_NVREG = _HIDDEN // 16
_CHUNK = 128


@functools.partial(jax.jit, static_argnums=(4, 5))
def _embed(idx, tf, vocab_table, type_table, n_rows, n_workers):
    per_w = n_rows // n_workers
    n_chunks = per_w // _CHUNK
    nc = plsc.get_sparse_core_info().num_cores

    def body(idx_hbm, tf_hbm, vt_hbm, tt_hbm, out_hbm,
             idx_v, tf_v, rows_v, tt_v, sem):
        wid = lax.axis_index("s") * nc + lax.axis_index("c")
        base = wid * per_w
        pltpu.sync_copy(tt_hbm, tt_v)
        t0 = [tt_v[0, pl.ds(16 * k, 16)] for k in range(_NVREG)]
        dt = [tt_v[1, pl.ds(16 * k, 16)] - t0[k] for k in range(_NVREG)]

        def chunk_body(c, carry):
            off = base + c * _CHUNK
            pltpu.sync_copy(idx_hbm.at[pl.ds(off, _CHUNK)], idx_v)
            pltpu.sync_copy(tf_hbm.at[pl.ds(off, _CHUNK)], tf_v)
            pltpu.async_copy(vt_hbm.at[idx_v], rows_v, sem).wait()

            def row_body(r, rcarry):
                tsp = plsc.load_gather(tf_v, [jnp.full((16,), r, jnp.int32)])
                for k in range(_NVREG):
                    sl = pl.ds(16 * k, 16)
                    rows_v[r, sl] = rows_v[r, sl] + (t0[k] + tsp * dt[k])
                return rcarry

            lax.fori_loop(0, _CHUNK, row_body, 0)
            pltpu.sync_copy(rows_v, out_hbm.at[pl.ds(off, _CHUNK)])
            return carry

        lax.fori_loop(0, n_chunks, chunk_body, 0)

    return pl.kernel(
        body,
        out_type=jax.ShapeDtypeStruct((n_rows, _HIDDEN), jnp.float32),
        mesh=plsc.VectorSubcoreMesh(core_axis_name="c", subcore_axis_name="s"),
        compiler_params=pltpu.CompilerParams(needs_layout_passes=False),
        scratch_types=[
            pltpu.VMEM((_CHUNK,), jnp.int32),
            pltpu.VMEM((_CHUNK,), jnp.float32),
            pltpu.VMEM((_CHUNK, _HIDDEN), jnp.float32),
            pltpu.VMEM((2, _HIDDEN), jnp.float32),
            pltpu.SemaphoreType.DMA,
        ],
    )(idx, tf, vocab_table, type_table)


def kernel(vocab, type, vocab_table, type_table):
    b, l = vocab.shape
    n_rows = b * l
    info = plsc.get_sparse_core_info()
    n_workers = info.num_cores * info.num_subcores
    idx = vocab.reshape(n_rows)
    tf = type.reshape(n_rows).astype(jnp.float32)
    out = _embed(idx, tf, vocab_table, type_table, n_rows, n_workers)
    return out.reshape(b, l, _HIDDEN)


# speedup vs baseline: 5.9037x; 1.4712x over previous
import functools

import jax
import jax.numpy as jnp
from jax import lax
from jax.experimental import pallas as pl
from jax.experimental.pallas import tpu as pltpu
from jax.experimental.pallas import tpu_sc as plsc

_HIDDEN = 128
_NVREG = _HIDDEN // 16
_CHUNK = 128
_NBUF = 5
_GROUP = 16


@functools.partial(jax.jit, static_argnums=(4, 5))
def _embed(idx, tf, vocab_table, type_table, n_rows, n_workers):
    per_w = n_rows // n_workers
    n_chunks = per_w // _CHUNK
    n_outer = n_chunks // _NBUF
    nc = plsc.get_sparse_core_info().num_cores

    def body(idx_hbm, tf_hbm, vt_hbm, tt_hbm, out_hbm, *refs):
        idx_v, tf_v, tt_v = refs[0], refs[1], refs[2]
        rows = refs[3:3 + _NBUF]
        gsem = refs[3 + _NBUF:3 + 2 * _NBUF]
        wsem = refs[3 + 2 * _NBUF:3 + 3 * _NBUF]

        wid = lax.axis_index("s") * nc + lax.axis_index("c")
        base = wid * per_w

        pltpu.sync_copy(idx_hbm.at[pl.ds(base, per_w)], idx_v)
        pltpu.sync_copy(tf_hbm.at[pl.ds(base, per_w)], tf_v)
        pltpu.sync_copy(tt_hbm, tt_v)
        t0 = [tt_v[0, pl.ds(16 * k, 16)] for k in range(_NVREG)]
        dt = [tt_v[1, pl.ds(16 * k, 16)] - t0[k] for k in range(_NVREG)]

        def gather_args(c, b):
            return (vt_hbm.at[idx_v.at[pl.ds(c * _CHUNK, _CHUNK)]], rows[b],
                    gsem[b])

        def writeback_args(c, b):
            return (rows[b], out_hbm.at[pl.ds(base + c * _CHUNK, _CHUNK)],
                    wsem[b])

        def gather(c, b):
            pltpu.async_copy(*gather_args(c, b))

        def gather_wait(c, b):
            pltpu.make_async_copy(*gather_args(c, b)).wait()

        def writeback(c, b):
            pltpu.async_copy(*writeback_args(c, b))

        def writeback_wait(c, b):
            pltpu.make_async_copy(*writeback_args(c, b)).wait()

        for c in range(_NBUF - 2):
            gather(c, c)

        def compute(buf, c):
            def group_body(g, carry):
                tvec = tf_v[pl.ds(c * _CHUNK + g * _GROUP, _GROUP)]
                for j in range(_GROUP):
                    tsp = tvec.at[jnp.full((16,), j, jnp.int32)].get(
                        mode="promise_in_bounds")
                    r = g * _GROUP + j
                    for k in range(_NVREG):
                        sl = pl.ds(16 * k, 16)
                        buf[r, sl] = buf[r, sl] + (t0[k] + tsp * dt[k])
                return carry

            lax.fori_loop(0, _CHUNK // _GROUP, group_body, 0)

        def outer_body(gi, carry):
            for b in range(_NBUF):
                c = gi * _NBUF + b
                gather_wait(c, b)
                compute(rows[b], c)
                writeback(c, b)
                br = (b + 3) % _NBUF

                @pl.when(c >= 2)
                def _():
                    writeback_wait(c - 2, br)

                @pl.when(c <= n_chunks - 1 - (_NBUF - 2))
                def _():
                    gather(c + _NBUF - 2, br)

            return carry

        lax.fori_loop(0, n_outer, outer_body, 0)
        writeback_wait(n_chunks - 2, (n_chunks - 2) % _NBUF)
        writeback_wait(n_chunks - 1, (n_chunks - 1) % _NBUF)

    return pl.kernel(
        body,
        out_type=jax.ShapeDtypeStruct((n_rows, _HIDDEN), jnp.float32),
        mesh=plsc.VectorSubcoreMesh(core_axis_name="c", subcore_axis_name="s"),
        compiler_params=pltpu.CompilerParams(needs_layout_passes=False),
        scratch_types=(
            [
                pltpu.VMEM((per_w,), jnp.int32),
                pltpu.VMEM((per_w,), jnp.float32),
                pltpu.VMEM((2, _HIDDEN), jnp.float32),
            ]
            + [pltpu.VMEM((_CHUNK, _HIDDEN), jnp.float32)] * _NBUF
            + [pltpu.SemaphoreType.DMA] * (2 * _NBUF)
        ),
    )(idx, tf, vocab_table, type_table)


def kernel(vocab, type, vocab_table, type_table):
    b, l = vocab.shape
    n_rows = b * l
    info = plsc.get_sparse_core_info()
    n_workers = info.num_cores * info.num_subcores
    idx = vocab.reshape(n_rows)
    tf = type.reshape(n_rows).astype(jnp.float32)
    out = _embed(idx, tf, vocab_table, type_table, n_rows, n_workers)
    return out.reshape(b, l, _HIDDEN)
